# baseline (speedup 1.0000x reference)
import jax
import jax.numpy as jnp
from jax.experimental import pallas as pl
from jax.experimental.pallas import tpu as pltpu

TOKENS = 16384
D_MODEL = 2048
HIDDEN = 256
N_EXPERTS = 64
TOP_K = 8
BT = 2048
NX = 4
DH = D_MODEL // NX
RB = 256


def _gating_kernel(xa_ref, xb_ref, xc_ref, xd_ref, w1_ref, b1_ref, w2_ref,
                   b2_ref, out_ref):
    x_refs = (xa_ref, xb_ref, xc_ref, xd_ref)
    for r in range(BT // RB):
        rows = pl.ds(r * RB, RB)
        h = b1_ref[...].astype(jnp.float32)
        acc = None
        for k, xr in enumerate(x_refs):
            part = jnp.dot(
                xr[rows, :],
                w1_ref[k * DH:(k + 1) * DH, :],
                preferred_element_type=jnp.float32,
            )
            acc = part if acc is None else acc + part
        h = acc + b1_ref[...]
        h = 0.5 * h * (1.0 + jax.lax.erf(h * 0.7071067811865476))
        logits = (
            jnp.dot(h, w2_ref[...], preferred_element_type=jnp.float32)
            + b2_ref[...]
        )

        work = logits
        for _ in range(TOP_K - 1):
            m = jnp.max(work, axis=-1, keepdims=True)
            work = jnp.where(work >= m, -jnp.inf, work)
        t = jnp.max(work, axis=-1, keepdims=True)
        sel = logits >= t

        mx = jnp.max(logits, axis=-1, keepdims=True)
        e = jnp.where(sel, jnp.exp(logits - mx), 0.0)
        z = jnp.sum(e, axis=-1, keepdims=True)
        out_ref[rows, :] = e / z


@jax.jit
def kernel(x, W1, b1, W2, b2):
    w1t = W1.T
    w2t = W2.T
    b1r = b1.reshape(1, HIDDEN)
    b2r = b2.reshape(1, N_EXPERTS)

    grid = (TOKENS // BT,)
    sparse_weights = pl.pallas_call(
        _gating_kernel,
        grid=grid,
        in_specs=[
            pl.BlockSpec((BT, DH), lambda i: (i, 0)),
            pl.BlockSpec((BT, DH), lambda i: (i, 1)),
            pl.BlockSpec((BT, DH), lambda i: (i, 2)),
            pl.BlockSpec((BT, DH), lambda i: (i, 3)),
            pl.BlockSpec((D_MODEL, HIDDEN), lambda i: (0, 0)),
            pl.BlockSpec((1, HIDDEN), lambda i: (0, 0)),
            pl.BlockSpec((HIDDEN, N_EXPERTS), lambda i: (0, 0)),
            pl.BlockSpec((1, N_EXPERTS), lambda i: (0, 0)),
        ],
        out_specs=pl.BlockSpec((BT, N_EXPERTS), lambda i: (i, 0)),
        out_shape=jax.ShapeDtypeStruct((TOKENS, N_EXPERTS), jnp.float32),
        compiler_params=pltpu.CompilerParams(
            dimension_semantics=("parallel",),
        ),
    )(x, x, x, x, w1t, b1r, w2t, b2r)

    aux_loss = jnp.asarray(0.0, dtype=jnp.float32)
    return (sparse_weights, aux_loss)


# speedup vs baseline: 1.0820x; 1.0820x over previous
import jax
import jax.numpy as jnp
from jax.experimental import pallas as pl
from jax.experimental.pallas import tpu as pltpu

TOKENS = 16384
D_MODEL = 2048
HIDDEN = 256
N_EXPERTS = 64
TOP_K = 8
BT = 2048
NX = 4
DH = D_MODEL // NX
RB = 256


def _gating_kernel(xa_ref, xb_ref, xc_ref, xd_ref, w1_ref, b1_ref, w2_ref,
                   b2_ref, out_ref):
    x_refs = (xa_ref, xb_ref, xc_ref, xd_ref)
    acc = None
    for k, xr in enumerate(x_refs):
        part = jnp.dot(
            xr[...],
            w1_ref[k * DH:(k + 1) * DH, :],
            preferred_element_type=jnp.float32,
        )
        acc = part if acc is None else acc + part
    h = acc + b1_ref[...]
    h = 0.5 * h * (1.0 + jax.lax.erf(h * 0.7071067811865476))
    all_logits = (
        jnp.dot(h, w2_ref[...], preferred_element_type=jnp.float32)
        + b2_ref[...]
    )

    for r in range(BT // RB):
        rows = pl.ds(r * RB, RB)
        logits = all_logits[r * RB:(r + 1) * RB, :]

        work = logits
        for _ in range(TOP_K - 1):
            m = jnp.max(work, axis=-1, keepdims=True)
            work = jnp.where(work >= m, -jnp.inf, work)
        t = jnp.max(work, axis=-1, keepdims=True)
        sel = logits >= t

        mx = jnp.max(logits, axis=-1, keepdims=True)
        e = jnp.where(sel, jnp.exp(logits - mx), 0.0)
        z = jnp.sum(e, axis=-1, keepdims=True)
        out_ref[rows, :] = e / z


@jax.jit
def kernel(x, W1, b1, W2, b2):
    w1t = W1.T
    w2t = W2.T
    b1r = b1.reshape(1, HIDDEN)
    b2r = b2.reshape(1, N_EXPERTS)

    grid = (TOKENS // BT,)
    sparse_weights = pl.pallas_call(
        _gating_kernel,
        grid=grid,
        in_specs=[
            pl.BlockSpec((BT, DH), lambda i: (i, 0)),
            pl.BlockSpec((BT, DH), lambda i: (i, 1)),
            pl.BlockSpec((BT, DH), lambda i: (i, 2)),
            pl.BlockSpec((BT, DH), lambda i: (i, 3)),
            pl.BlockSpec((D_MODEL, HIDDEN), lambda i: (0, 0)),
            pl.BlockSpec((1, HIDDEN), lambda i: (0, 0)),
            pl.BlockSpec((HIDDEN, N_EXPERTS), lambda i: (0, 0)),
            pl.BlockSpec((1, N_EXPERTS), lambda i: (0, 0)),
        ],
        out_specs=pl.BlockSpec((BT, N_EXPERTS), lambda i: (i, 0)),
        out_shape=jax.ShapeDtypeStruct((TOKENS, N_EXPERTS), jnp.float32),
        compiler_params=pltpu.CompilerParams(
            dimension_semantics=("parallel",),
        ),
    )(x, x, x, x, w1t, b1r, w2t, b2r)

    aux_loss = jnp.asarray(0.0, dtype=jnp.float32)
    return (sparse_weights, aux_loss)


# speedup vs baseline: 1.1170x; 1.0324x over previous
import jax
import jax.numpy as jnp
from jax.experimental import pallas as pl
from jax.experimental.pallas import tpu as pltpu

TOKENS = 16384
D_MODEL = 2048
HIDDEN = 256
N_EXPERTS = 64
TOP_K = 8
BT = 2048
NX = 2
DH = D_MODEL // NX
RB = 256


def _gating_kernel(xa_ref, xb_ref, w1_ref, b1_ref, w2_ref,
                   b2_ref, out_ref):
    x_refs = (xa_ref, xb_ref)
    acc = None
    for k, xr in enumerate(x_refs):
        part = jnp.dot(
            xr[...],
            w1_ref[k * DH:(k + 1) * DH, :],
            preferred_element_type=jnp.float32,
        )
        acc = part if acc is None else acc + part
    h = acc + b1_ref[...]
    h = 0.5 * h * (1.0 + jax.lax.erf(h * 0.7071067811865476))
    all_logits = (
        jnp.dot(h, w2_ref[...], preferred_element_type=jnp.float32)
        + b2_ref[...]
    )

    for r in range(BT // RB):
        rows = pl.ds(r * RB, RB)
        logits = all_logits[r * RB:(r + 1) * RB, :]

        work = logits
        for _ in range(TOP_K - 1):
            m = jnp.max(work, axis=-1, keepdims=True)
            work = jnp.where(work >= m, -jnp.inf, work)
        t = jnp.max(work, axis=-1, keepdims=True)
        sel = logits >= t

        mx = jnp.max(logits, axis=-1, keepdims=True)
        e = jnp.where(sel, jnp.exp(logits - mx), 0.0)
        z = jnp.sum(e, axis=-1, keepdims=True)
        out_ref[rows, :] = e / z


@jax.jit
def kernel(x, W1, b1, W2, b2):
    w1t = W1.T
    w2t = W2.T
    b1r = b1.reshape(1, HIDDEN)
    b2r = b2.reshape(1, N_EXPERTS)

    grid = (TOKENS // BT,)
    sparse_weights = pl.pallas_call(
        _gating_kernel,
        grid=grid,
        in_specs=[
            pl.BlockSpec((BT, DH), lambda i: (i, 0)),
            pl.BlockSpec((BT, DH), lambda i: (i, 1)),
            pl.BlockSpec((D_MODEL, HIDDEN), lambda i: (0, 0)),
            pl.BlockSpec((1, HIDDEN), lambda i: (0, 0)),
            pl.BlockSpec((HIDDEN, N_EXPERTS), lambda i: (0, 0)),
            pl.BlockSpec((1, N_EXPERTS), lambda i: (0, 0)),
        ],
        out_specs=pl.BlockSpec((BT, N_EXPERTS), lambda i: (i, 0)),
        out_shape=jax.ShapeDtypeStruct((TOKENS, N_EXPERTS), jnp.float32),
        compiler_params=pltpu.CompilerParams(
            dimension_semantics=("parallel",),
        ),
    )(x, x, w1t, b1r, w2t, b2r)

    aux_loss = jnp.asarray(0.0, dtype=jnp.float32)
    return (sparse_weights, aux_loss)
